# baseline (speedup 1.0000x reference)
import jax
import jax.numpy as jnp
from jax.experimental import pallas as pl
from jax.experimental.pallas import tpu as pltpu

B, N, C, W, H = 8, 16, 64, 32, 32
WH = W * H
TS = 64


def _assemble_kernel(x_ref, t_ref, emb_ref, out_ref, tv_ref):
    for n in range(N):
        row = emb_ref[t_ref[n], :]
        tv_ref[n] = jax.lax.broadcast_in_dim(row, (TS, WH), (0,))

    for n in range(N):
        out_ref[0, n, :C, :] = x_ref[0, n]
        out_ref[0, n, C:, :] = tv_ref[n]


def kernel(x, t, emb_table):
    x2 = x.reshape(B, N, C, WH)
    out = pl.pallas_call(
        _assemble_kernel,
        grid=(B,),
        in_specs=[
            pl.BlockSpec((1, N, C, WH), lambda i: (i, 0, 0, 0)),
            pl.BlockSpec(memory_space=pltpu.SMEM),
            pl.BlockSpec(emb_table.shape, lambda i: (0, 0)),
        ],
        out_specs=pl.BlockSpec((1, N, C + TS, WH), lambda i: (i, 0, 0, 0)),
        out_shape=jax.ShapeDtypeStruct((B, N, C + TS, WH), x.dtype),
        scratch_shapes=[pltpu.VMEM((N, TS, WH), x.dtype)],
        compiler_params=pltpu.CompilerParams(
            dimension_semantics=("parallel",)),
    )(x2, t.astype(jnp.int32), emb_table)
    return out.reshape(B, N, C + TS, W, H)
